# baseline (speedup 1.0000x reference)
import functools

import jax
import jax.numpy as jnp
from jax import lax
from jax.experimental import pallas as pl
from jax.experimental.pallas import tpu as pltpu
from jax.experimental.pallas import tpu_sc as plsc

_NC = 2
_NS = 16
_NW = _NC * _NS

_BATCH = 4096
_HIST = 50

_ROWS_BLK = 4000


def _compress_body(x_ref, w_ref, b_ref, o_ref):
    y = lax.dot_general(x_ref[...], w_ref[...],
                        (((1,), (1,)), ((), ())),
                        preferred_element_type=jnp.float32)
    o_ref[:, 0:32] = jnp.clip(y + b_ref[...], -1.0, 1.0)


def _compress_table(table, W, b):
    n_rows, pre = table.shape
    comp = W.shape[0]
    grid = n_rows // _ROWS_BLK
    return pl.pallas_call(
        _compress_body,
        grid=(grid,),
        in_specs=[
            pl.BlockSpec((_ROWS_BLK, pre), lambda i: (i, 0)),
            pl.BlockSpec((comp, pre), lambda i: (0, 0)),
            pl.BlockSpec((1, comp), lambda i: (0, 0)),
        ],
        out_specs=pl.BlockSpec((_ROWS_BLK, pre), lambda i: (i, 0)),
        out_shape=jax.ShapeDtypeStruct((n_rows, pre), jnp.float32),
    )(table, W, b.reshape(1, comp))


_HG = 2
_NHG = _HIST // _HG
_GLEN = _HG * 128


def _sc_gather_body(comp_hbm, idx_hbm, out_hbm, idx_v, rows_g, p_v,
                    gsem, wsem):
    c = lax.axis_index("c")
    s = lax.axis_index("s")
    wid = s * _NC + c
    pltpu.sync_copy(idx_hbm.at[wid], idx_v)

    def gather(u, slot):
        return pltpu.make_async_copy(
            comp_hbm.at[idx_v.at[u]],
            rows_g.at[slot, pl.ds(lax.rem(u, _HG) * 128, 128)],
            gsem.at[slot])

    def fire_group(g, slot):
        for k in range(_HG):
            gather(g * _HG + k, slot).start()

    def drain_group(g, slot):
        for k in range(_HG):
            gather(g * _HG + k, slot).wait()

    def write(g, slot):
        return pltpu.make_async_copy(
            p_v.at[slot],
            out_hbm.at[pl.ds(g * _HG, _HG), slice(None), pl.ds(wid, 1)],
            wsem.at[slot])

    lanes = lax.iota(jnp.int32, 16)

    ridx_all = [[lanes + (hh * 128 + 16 * k) for k in range(8)]
                for hh in range(_HG)]

    def repack(slot):
        src2d = rows_g.at[slot]

        @plsc.parallel_loop(0, 32, unroll=8)
        def _(t):
            tr = t // 8
            cr = lax.rem(t, 8)
            cc = 8 * tr + cr
            ccv = jnp.full((16,), 0, jnp.int32) + cc
            for hh in range(_HG):
                for k in range(8):
                    vec = plsc.load_gather(src2d, [ridx_all[hh][k], ccv])
                    p_v[slot, hh, tr, 0, cr, pl.ds(16 * k, 16)] = vec

    fire_group(0, 0)

    def step(g, carry):
        slot = lax.rem(g, 2)
        nslot = lax.rem(g + 1, 2)

        @pl.when(g + 1 < _NHG)
        def _():
            fire_group(g + 1, nslot)

        drain_group(g, slot)

        @pl.when(g >= 2)
        def _():
            write(g - 2, slot).wait()
        repack(slot)
        write(g, slot).start()
        return carry

    lax.fori_loop(0, _NHG, step, 0)
    write(_NHG - 2, lax.rem(_NHG - 2, 2)).wait()
    write(_NHG - 1, lax.rem(_NHG - 1, 2)).wait()


def _sc_gather(comp_tbl4, idx_t):
    mesh = plsc.VectorSubcoreMesh(core_axis_name="c", subcore_axis_name="s")
    fn = functools.partial(
        pl.kernel,
        out_type=jax.ShapeDtypeStruct((_HIST, 4, _NW, 8, 128), jnp.float32),
        mesh=mesh,
        scratch_types=[
            pltpu.VMEM((_HIST, 128), jnp.int32),
            pltpu.VMEM((2, _GLEN, 32), jnp.float32),
            pltpu.VMEM((2, _HG, 4, 1, 8, 128), jnp.float32),
            pltpu.SemaphoreType.DMA((2,)),
            pltpu.SemaphoreType.DMA((2,)),
        ],
        compiler_params=pltpu.CompilerParams(use_tc_tiling_on_sc=False,
                                             needs_layout_passes=False),
    )(_sc_gather_body)
    return fn(comp_tbl4, idx_t)


def kernel(indices, table, W, b):
    batch, hist = indices.shape
    n_rows, pre = table.shape
    comp_dim = W.shape[0]
    comp_table = _compress_table(table, W, b)
    comp_tbl4 = comp_table.reshape(n_rows * (pre // comp_dim), comp_dim)
    idx4 = indices.astype(jnp.int32) * 4
    idx_t = idx4.reshape(_NW, 128, hist).transpose(0, 2, 1)
    out5 = _sc_gather(comp_tbl4, idx_t)
    out = out5.transpose(2, 4, 0, 1, 3).reshape(batch, hist, comp_dim)
    return out


# speedup vs baseline: 1.0747x; 1.0747x over previous
import functools

import jax
import jax.numpy as jnp
from jax import lax
from jax.experimental import pallas as pl
from jax.experimental.pallas import tpu as pltpu
from jax.experimental.pallas import tpu_sc as plsc

_NC = 2
_NS = 16
_NW = _NC * _NS

_BATCH = 4096
_HIST = 50

_ROWS_BLK = 4000


def _compress_body(x_ref, w_ref, b_ref, o_ref):
    y = lax.dot_general(x_ref[...], w_ref[...],
                        (((1,), (1,)), ((), ())),
                        preferred_element_type=jnp.float32)
    o_ref[:, 0:32] = jnp.clip(y + b_ref[...], -1.0, 1.0)


def _compress_table(table, W, b):
    n_rows, pre = table.shape
    comp = W.shape[0]
    grid = n_rows // _ROWS_BLK
    return pl.pallas_call(
        _compress_body,
        grid=(grid,),
        in_specs=[
            pl.BlockSpec((_ROWS_BLK, pre), lambda i: (i, 0)),
            pl.BlockSpec((comp, pre), lambda i: (0, 0)),
            pl.BlockSpec((1, comp), lambda i: (0, 0)),
        ],
        out_specs=pl.BlockSpec((_ROWS_BLK, pre), lambda i: (i, 0)),
        out_shape=jax.ShapeDtypeStruct((n_rows, pre), jnp.float32),
    )(table, W, b.reshape(1, comp))


_HG = 2
_NHG = _HIST // _HG
_GLEN = _HG * 128


def _sc_gather_body(comp_hbm, idx_hbm, out_hbm, idx_v, rows_g, p_v,
                    gsem, wsem):
    c = lax.axis_index("c")
    s = lax.axis_index("s")
    wid = s * _NC + c
    pltpu.sync_copy(idx_hbm.at[wid], idx_v)

    def gather(u, slot):
        return pltpu.make_async_copy(
            comp_hbm.at[idx_v.at[u]],
            rows_g.at[slot, pl.ds(lax.rem(u, _HG) * 128, 128)],
            gsem.at[slot])

    def fire_group(g, slot):
        for k in range(_HG):
            gather(g * _HG + k, slot).start()

    def drain_group(g, slot):
        for k in range(_HG):
            gather(g * _HG + k, slot).wait()

    def write(g, slot):
        return pltpu.make_async_copy(
            p_v.at[slot],
            out_hbm.at[pl.ds(g * _HG, _HG), slice(None), pl.ds(wid, 1)],
            wsem.at[slot])

    lanes = lax.iota(jnp.int32, 16)

    ridx_all = [[lanes + (hh * 128 + 16 * k) for k in range(8)]
                for hh in range(_HG)]

    def repack(slot):
        src2d = rows_g.at[slot]

        @plsc.parallel_loop(0, 32, unroll=4)
        def _(t):
            tr = t // 8
            cr = lax.rem(t, 8)
            cc = 8 * tr + cr
            ccv = jnp.full((16,), 0, jnp.int32) + cc
            for hh in range(_HG):
                for k in range(8):
                    vec = plsc.load_gather(src2d, [ridx_all[hh][k], ccv])
                    p_v[slot, hh, tr, 0, cr, pl.ds(16 * k, 16)] = vec

    fire_group(0, 0)

    def step(g, carry):
        slot = lax.rem(g, 2)
        nslot = lax.rem(g + 1, 2)

        @pl.when(g + 1 < _NHG)
        def _():
            fire_group(g + 1, nslot)

        drain_group(g, slot)

        @pl.when(g >= 2)
        def _():
            write(g - 2, slot).wait()
        repack(slot)
        write(g, slot).start()
        return carry

    lax.fori_loop(0, _NHG, step, 0)
    write(_NHG - 2, lax.rem(_NHG - 2, 2)).wait()
    write(_NHG - 1, lax.rem(_NHG - 1, 2)).wait()


def _sc_gather(comp_tbl4, idx_t):
    mesh = plsc.VectorSubcoreMesh(core_axis_name="c", subcore_axis_name="s")
    fn = functools.partial(
        pl.kernel,
        out_type=jax.ShapeDtypeStruct((_HIST, 4, _NW, 8, 128), jnp.float32),
        mesh=mesh,
        scratch_types=[
            pltpu.VMEM((_HIST, 128), jnp.int32),
            pltpu.VMEM((2, _GLEN, 32), jnp.float32),
            pltpu.VMEM((2, _HG, 4, 1, 8, 128), jnp.float32),
            pltpu.SemaphoreType.DMA((2,)),
            pltpu.SemaphoreType.DMA((2,)),
        ],
        compiler_params=pltpu.CompilerParams(use_tc_tiling_on_sc=False,
                                             needs_layout_passes=False),
    )(_sc_gather_body)
    return fn(comp_tbl4, idx_t)


def kernel(indices, table, W, b):
    batch, hist = indices.shape
    n_rows, pre = table.shape
    comp_dim = W.shape[0]
    comp_table = _compress_table(table, W, b)
    comp_tbl4 = comp_table.reshape(n_rows * (pre // comp_dim), comp_dim)
    idx4 = indices.astype(jnp.int32) * 4
    idx_t = idx4.reshape(_NW, 128, hist).transpose(0, 2, 1)
    out5 = _sc_gather(comp_tbl4, idx_t)
    out = out5.transpose(2, 4, 0, 1, 3).reshape(batch, hist, comp_dim)
    return out
